# baseline (speedup 1.0000x reference)
import functools

import jax
import jax.numpy as jnp
from jax import lax
from jax.experimental import pallas as pl
from jax.experimental.pallas import tpu as pltpu
from jax.experimental.pallas import tpu_sc as plsc

NC = 2
NS = 16
CB = 128


def _tc_matmul(x, w):
  m, _ = x.shape
  h = w.shape[1]

  def body(x_ref, w_ref, o_ref):
    o_ref[...] = jnp.dot(x_ref[...], w_ref[...],
                         preferred_element_type=jnp.float32)

  return pl.pallas_call(
      body,
      out_shape=jax.ShapeDtypeStruct((m, h), jnp.float32),
  )(x, w)


def _tc_combine_relu_matmul(p, w):
  _, n_pad, _ = p.shape
  h = w.shape[1]

  def body(p_ref, w_ref, o_ref):
    h1 = jnp.maximum(p_ref[0] + p_ref[1], 0.0)
    o_ref[...] = jnp.dot(h1, w_ref[...], preferred_element_type=jnp.float32)

  return pl.pallas_call(
      body,
      out_shape=jax.ShapeDtypeStruct((n_pad, h), jnp.float32),
  )(p, w)


def _tc_combine_tanh(q, n_out):
  _, _, h = q.shape

  def body(q_ref, o_ref):
    o_ref[...] = jnp.tanh(q_ref[0, :n_out] + q_ref[1, :n_out])

  return pl.pallas_call(
      body,
      out_shape=jax.ShapeDtypeStruct((n_out, h), jnp.float32),
  )(q)


@functools.lru_cache(maxsize=None)
def _make_spmm(n_pad, h, ept):
  rows_pt = n_pad // NS
  bpt = ept // CB
  nch = 3
  rch = bpt // nch
  mesh = plsc.VectorSubcoreMesh(core_axis_name="c", subcore_axis_name="s")

  @functools.partial(
      pl.kernel,
      out_type=jax.ShapeDtypeStruct((NC, n_pad, h), jnp.float32),
      mesh=mesh,
      compiler_params=pltpu.CompilerParams(use_tc_tiling_on_sc=False),
      scratch_types=[
          pltpu.VMEM_SHARED((n_pad, h), jnp.float32),
          pltpu.VMEM_SHARED((n_pad, h), jnp.float32),
          pltpu.VMEM((3, CB, h), jnp.float32),
          pltpu.VMEM((2, rch, CB), jnp.int32),
          pltpu.VMEM((2, rch, CB), jnp.int32),
          pltpu.VMEM((2, rch * CB), jnp.float32),
          pltpu.SemaphoreType.DMA,
          pltpu.SemaphoreType.DMA,
          pltpu.SemaphoreType.DMA,
          pltpu.SemaphoreType.DMA,
          pltpu.SemaphoreType.DMA,
          pltpu.SemaphoreType.DMA,
          pltpu.SemaphoreType.DMA,
      ],
  )
  def spmm(feat_hbm, src_hbm, dst_hbm, w_hbm, out_hbm,
           acc, sfeat, rows, sidx, didx, wbuf, lsem, gs0, gs1, gs2,
           ss0, ss1, ss2):
    c = lax.axis_index("c")
    s = lax.axis_index("s")
    wid = s * NC + c

    def load_chunk(ci):
      slot = ci % 2
      return (
          pltpu.async_copy(
              src_hbm.at[pl.ds(wid * bpt + ci * rch, rch)],
              sidx.at[slot], lsem),
          pltpu.async_copy(
              dst_hbm.at[pl.ds(wid * bpt + ci * rch, rch)],
              didx.at[slot], lsem),
          pltpu.async_copy(
              w_hbm.at[pl.ds(wid * ept + ci * rch * CB, rch * CB)],
              wbuf.at[slot], lsem),
      )

    lds = load_chunk(0)

    @pl.loop(0, CB)
    def _(r):
      for cb in range(h // 16):
        rows[0, r, pl.ds(cb * 16, 16)] = jnp.zeros((16,), jnp.float32)

    @pl.loop(0, rows_pt // CB)
    def _(j):
      rb = s * rows_pt + j * CB
      pltpu.sync_copy(rows.at[0], acc.at[pl.ds(rb, CB)])
      pltpu.sync_copy(feat_hbm.at[pl.ds(rb, CB)], sfeat.at[pl.ds(rb, CB)])

    for ld in lds:
      ld.wait()

    gsems = (gs0, gs1, gs2)
    ssems = (ss0, ss1, ss2)

    def start_gather(slot, kl, b):
      pltpu.async_copy(sfeat.at[sidx.at[slot, kl]], rows.at[b], gsems[b])

    def wait_gather(slot, kl, b):
      pltpu.make_async_copy(sfeat.at[sidx.at[slot, kl]], rows.at[b],
                            gsems[b]).wait()

    def scale(slot, kl, b):
      @pl.loop(0, CB // 16)
      def _(g):
        w16 = wbuf[slot, pl.ds(kl * CB + g * 16, 16)]
        ews = []
        for j in range(16):
          ews.append(lax.gather(
              w16, jnp.full((16, 1), j, jnp.int32),
              lax.GatherDimensionNumbers(
                  offset_dims=(), collapsed_slice_dims=(0,),
                  start_index_map=(0,)),
              slice_sizes=(1,),
              mode=lax.GatherScatterMode.PROMISE_IN_BOUNDS))
        for cb in range(h // 16):
          sl = pl.ds(cb * 16, 16)
          vals = [rows[b, g * 16 + j, sl] * ews[j] for j in range(16)]
          for j in range(16):
            rows[b, g * 16 + j, sl] = vals[j]

    def start_scatter(slot, kl, b):
      pltpu.async_copy(rows.at[b], acc.at[didx.at[slot, kl]], ssems[b],
                       add=True)

    def drain_scatter(slot, kl, b):
      pltpu.make_async_copy(rows.at[b], acc.at[didx.at[slot, kl]],
                            ssems[b]).wait()

    plsc.subcore_barrier()

    n3 = rch // 3
    for ci in range(nch):
      slot = ci % 2
      if ci + 1 < nch:
        lds = load_chunk(ci + 1)

      start_gather(slot, 0, 0)
      start_gather(slot, 1, 1)

      @pl.loop(0, n3)
      def _(i3):
        for b in range(3):
          kl = i3 * 3 + b
          wait_gather(slot, kl, b)
          scale(slot, kl, b)
          start_scatter(slot, kl, b)
          b2 = (b + 2) % 3
          if b == 0:
            @pl.when(i3 >= 1)
            def _():
              drain_scatter(slot, kl - 1, b2)
            start_gather(slot, kl + 2, b2)
          else:
            @pl.when(i3 < n3 - 1)
            def _():
              drain_scatter(slot, kl - 1, b2)
              start_gather(slot, kl + 2, b2)

      drain_scatter(slot, rch - 3, 0)
      drain_scatter(slot, rch - 2, 1)
      drain_scatter(slot, rch - 1, 2)
      if ci + 1 < nch:
        for ld in lds:
          ld.wait()

    plsc.subcore_barrier()

    @pl.loop(0, rows_pt // CB)
    def _(j):
      rb = s * rows_pt + j * CB
      pltpu.sync_copy(acc.at[pl.ds(rb, CB)], out_hbm.at[c, pl.ds(rb, CB)])

  return spmm


def kernel(x, edge_index, edge_weight, W1, W2):
  n = x.shape[0]
  e = edge_index.shape[1]

  ept = -(-e // (NC * NS * CB * 9)) * (CB * 9)
  e_pad = NC * NS * ept
  n_pad = -(-n // (NS * CB)) * (NS * CB)

  src = edge_index[0].astype(jnp.int32)
  dst = edge_index[1].astype(jnp.int32)
  w = edge_weight.astype(jnp.float32)
  if e_pad != e:
    zpad_i = jnp.zeros((e_pad - e,), jnp.int32)
    src = jnp.concatenate([src, zpad_i])
    dst = jnp.concatenate([dst, zpad_i])
    w = jnp.concatenate([w, jnp.zeros((e_pad - e,), jnp.float32)])
  src = src.reshape(e_pad // CB, CB)
  dst = dst.reshape(e_pad // CB, CB)

  if n_pad != n:
    x = jnp.concatenate(
        [x, jnp.zeros((n_pad - n, x.shape[1]), jnp.float32)])
  xw1 = _tc_matmul(x, W1)
  p = _make_spmm(n_pad, W1.shape[1], ept)(xw1, src, dst, w)
  y2 = _tc_combine_relu_matmul(p, W2)
  q = _make_spmm(n_pad, W2.shape[1], ept)(y2, src, dst, w)
  return _tc_combine_tanh(q, n)


# speedup vs baseline: 1.0318x; 1.0318x over previous
import functools

import jax
import jax.numpy as jnp
from jax import lax
from jax.experimental import pallas as pl
from jax.experimental.pallas import tpu as pltpu
from jax.experimental.pallas import tpu_sc as plsc

NC = 2
NS = 16
CB = 128


def _tc_matmul_split(x, w):
  m = x.shape[0]
  h = w.shape[1]
  hh = h // 2

  def body(x_ref, w_ref, o1_ref, o2_ref):
    xw = jnp.dot(x_ref[...], w_ref[...], preferred_element_type=jnp.float32)
    o1_ref[...] = xw[:, :hh]
    o2_ref[...] = xw[:, hh:]

  return pl.pallas_call(
      body,
      out_shape=[jax.ShapeDtypeStruct((m, hh), jnp.float32),
                 jax.ShapeDtypeStruct((m, hh), jnp.float32)],
  )(x, w)


def _tc_relu_matmul_split(p, w):
  _, n_pad, k = p.shape
  h = w.shape[1]
  hh = h // 2

  def body(p_ref, w_ref, o1_ref, o2_ref):
    h1l = jnp.maximum(p_ref[0], 0.0)
    h1r = jnp.maximum(p_ref[1], 0.0)
    y = (jnp.dot(h1l, w_ref[:k], preferred_element_type=jnp.float32)
         + jnp.dot(h1r, w_ref[k:], preferred_element_type=jnp.float32))
    o1_ref[...] = y[:, :hh]
    o2_ref[...] = y[:, hh:]

  return pl.pallas_call(
      body,
      out_shape=[jax.ShapeDtypeStruct((n_pad, hh), jnp.float32),
                 jax.ShapeDtypeStruct((n_pad, hh), jnp.float32)],
  )(p, w)


@functools.lru_cache(maxsize=None)
def _make_spmm_col(n_pad, h, ept, tanh_out):
  rows_pt = n_pad // NS
  bpt = ept // CB
  mesh = plsc.VectorSubcoreMesh(core_axis_name="c", subcore_axis_name="s")

  @functools.partial(
      pl.kernel,
      out_type=jax.ShapeDtypeStruct((NC, n_pad, h), jnp.float32),
      mesh=mesh,
      compiler_params=pltpu.CompilerParams(use_tc_tiling_on_sc=False),
      scratch_types=[
          pltpu.VMEM_SHARED((n_pad, h), jnp.float32),
          pltpu.VMEM_SHARED((n_pad, h), jnp.float32),
          pltpu.VMEM((3, CB, h), jnp.float32),
          pltpu.VMEM((bpt, CB), jnp.int32),
          pltpu.VMEM((bpt, CB), jnp.int32),
          pltpu.VMEM((ept,), jnp.float32),
          pltpu.SemaphoreType.DMA,
          pltpu.SemaphoreType.DMA,
          pltpu.SemaphoreType.DMA,
          pltpu.SemaphoreType.DMA,
          pltpu.SemaphoreType.DMA,
          pltpu.SemaphoreType.DMA,
          pltpu.SemaphoreType.DMA,
      ],
  )
  def spmm(feat0_hbm, feat1_hbm, src_hbm, dst_hbm, w_hbm, out_hbm,
           acc, sfeat, rows, sidx, didx, wbuf, lsem, gs0, gs1, gs2,
           ss0, ss1, ss2):
    c = lax.axis_index("c")
    s = lax.axis_index("s")

    bb = s * bpt
    ld_s = pltpu.async_copy(src_hbm.at[pl.ds(bb, bpt)], sidx, lsem)
    ld_d = pltpu.async_copy(dst_hbm.at[pl.ds(bb, bpt)], didx, lsem)
    ld_w = pltpu.async_copy(w_hbm.at[pl.ds(s * ept, ept)], wbuf, lsem)

    @pl.loop(0, CB)
    def _(r):
      for cb in range(h // 16):
        rows[0, r, pl.ds(cb * 16, 16)] = jnp.zeros((16,), jnp.float32)

    @pl.loop(0, rows_pt // CB)
    def _(j):
      rb = s * rows_pt + j * CB
      pltpu.sync_copy(rows.at[0], acc.at[pl.ds(rb, CB)])

      @pl.when(c == 0)
      def _():
        pltpu.sync_copy(feat0_hbm.at[pl.ds(rb, CB)],
                        sfeat.at[pl.ds(rb, CB)])

      @pl.when(c == 1)
      def _():
        pltpu.sync_copy(feat1_hbm.at[pl.ds(rb, CB)],
                        sfeat.at[pl.ds(rb, CB)])

    ld_s.wait()
    ld_d.wait()
    ld_w.wait()

    gsems = (gs0, gs1, gs2)
    ssems = (ss0, ss1, ss2)

    def start_gather(it, b):
      pltpu.async_copy(sfeat.at[sidx.at[it]], rows.at[b], gsems[b])

    def wait_gather(it, b):
      pltpu.make_async_copy(sfeat.at[sidx.at[it]], rows.at[b],
                            gsems[b]).wait()

    def scale(it, b):
      @pl.loop(0, CB // 16)
      def _(g):
        w16 = wbuf[pl.ds(it * CB + g * 16, 16)]
        ews = []
        for j in range(16):
          ews.append(lax.gather(
              w16, jnp.full((16, 1), j, jnp.int32),
              lax.GatherDimensionNumbers(
                  offset_dims=(), collapsed_slice_dims=(0,),
                  start_index_map=(0,)),
              slice_sizes=(1,),
              mode=lax.GatherScatterMode.PROMISE_IN_BOUNDS))
        for cb in range(h // 16):
          sl = pl.ds(cb * 16, 16)
          vals = [rows[b, g * 16 + j, sl] * ews[j] for j in range(16)]
          for j in range(16):
            rows[b, g * 16 + j, sl] = vals[j]

    def start_scatter(it, b):
      pltpu.async_copy(rows.at[b], acc.at[didx.at[it]], ssems[b], add=True)

    def drain_scatter(it, b):
      pltpu.make_async_copy(rows.at[b], acc.at[didx.at[it]],
                            ssems[b]).wait()

    plsc.subcore_barrier()
    start_gather(0, 0)
    start_gather(1, 1)

    n3 = bpt // 3

    @pl.loop(0, n3)
    def _(i3):
      for b in range(3):
        k = i3 * 3 + b
        wait_gather(k, b)
        scale(k, b)
        start_scatter(k, b)
        b2 = (b + 2) % 3
        if b == 0:
          @pl.when(i3 >= 1)
          def _():
            drain_scatter(k - 1, b2)
          start_gather(k + 2, b2)
        else:
          @pl.when(i3 < n3 - 1)
          def _():
            drain_scatter(k - 1, b2)
            start_gather(k + 2, b2)

    drain_scatter(bpt - 3, 0)
    drain_scatter(bpt - 2, 1)
    drain_scatter(bpt - 1, 2)
    plsc.subcore_barrier()

    @pl.loop(0, rows_pt // CB)
    def _(j):
      rb = s * rows_pt + j * CB
      if tanh_out:
        pltpu.sync_copy(acc.at[pl.ds(rb, CB)], rows.at[0])

        @pl.loop(0, CB // 8)
        def _(r8):
          for dr in range(8):
            r = r8 * 8 + dr
            for cb in range(h // 16):
              sl = pl.ds(cb * 16, 16)
              v = rows[0, r, sl]
              e2 = jnp.exp(v * 2.0)
              rows[0, r, sl] = 1.0 - 2.0 / (e2 + 1.0)

        pltpu.sync_copy(rows.at[0], out_hbm.at[c, pl.ds(rb, CB)])
      else:
        pltpu.sync_copy(acc.at[pl.ds(rb, CB)], out_hbm.at[c, pl.ds(rb, CB)])

  return spmm


def kernel(x, edge_index, edge_weight, W1, W2):
  n = x.shape[0]
  e = edge_index.shape[1]

  ept = -(-e // (NS * CB * 3)) * (CB * 3)
  e_pad = NS * ept
  n_pad = -(-n // (NS * CB)) * (NS * CB)

  src = edge_index[0].astype(jnp.int32)
  dst = edge_index[1].astype(jnp.int32)
  w = edge_weight.astype(jnp.float32)
  if e_pad != e:
    zpad_i = jnp.zeros((e_pad - e,), jnp.int32)
    src = jnp.concatenate([src, zpad_i])
    dst = jnp.concatenate([dst, zpad_i])
    w = jnp.concatenate([w, jnp.zeros((e_pad - e,), jnp.float32)])
  src = src.reshape(e_pad // CB, CB)
  dst = dst.reshape(e_pad // CB, CB)

  if n_pad != n:
    x = jnp.concatenate(
        [x, jnp.zeros((n_pad - n, x.shape[1]), jnp.float32)])

  h1w = W1.shape[1]
  h2w = W2.shape[1]

  xw_l, xw_r = _tc_matmul_split(x, W1)
  p = _make_spmm_col(n_pad, h1w // 2, ept, False)(
      xw_l, xw_r, src, dst, w)
  y2_l, y2_r = _tc_relu_matmul_split(p, W2)
  q = _make_spmm_col(n_pad, h2w // 2, ept, True)(
      y2_l, y2_r, src, dst, w)
  return jnp.concatenate([q[0], q[1]], axis=1)[:n]


# speedup vs baseline: 1.0868x; 1.0533x over previous
import functools

import jax
import jax.numpy as jnp
from jax import lax
from jax.experimental import pallas as pl
from jax.experimental.pallas import tpu as pltpu
from jax.experimental.pallas import tpu_sc as plsc

NC = 2
NS = 16
CB = 128


def _tc_matmul_split(x, w):
  m = x.shape[0]
  h = w.shape[1]
  hh = h // 2

  def body(x_ref, w_ref, o1_ref, o2_ref):
    xw = jnp.dot(x_ref[...], w_ref[...], preferred_element_type=jnp.float32)
    o1_ref[...] = xw[:, :hh]
    o2_ref[...] = xw[:, hh:]

  return pl.pallas_call(
      body,
      out_shape=[jax.ShapeDtypeStruct((m, hh), jnp.float32),
                 jax.ShapeDtypeStruct((m, hh), jnp.float32)],
  )(x, w)


def _tc_relu_matmul_split(p, w):
  _, n_pad, k = p.shape
  h = w.shape[1]
  hh = h // 2

  def body(p_ref, w_ref, o1_ref, o2_ref):
    h1l = jnp.maximum(p_ref[0], 0.0)
    h1r = jnp.maximum(p_ref[1], 0.0)
    y = (jnp.dot(h1l, w_ref[:k], preferred_element_type=jnp.float32)
         + jnp.dot(h1r, w_ref[k:], preferred_element_type=jnp.float32))
    o1_ref[...] = y[:, :hh]
    o2_ref[...] = y[:, hh:]

  return pl.pallas_call(
      body,
      out_shape=[jax.ShapeDtypeStruct((n_pad, hh), jnp.float32),
                 jax.ShapeDtypeStruct((n_pad, hh), jnp.float32)],
  )(p, w)


@functools.lru_cache(maxsize=None)
def _make_spmm_col(n_pad, h, ept, tanh_out):
  rows_pt = n_pad // NS
  bpt = ept // CB
  mesh = plsc.VectorSubcoreMesh(core_axis_name="c", subcore_axis_name="s")

  @functools.partial(
      pl.kernel,
      out_type=jax.ShapeDtypeStruct((NC, n_pad, h), jnp.float32),
      mesh=mesh,
      compiler_params=pltpu.CompilerParams(use_tc_tiling_on_sc=False),
      scratch_types=[
          pltpu.VMEM_SHARED((n_pad, h), jnp.float32),
          pltpu.VMEM_SHARED((n_pad, h), jnp.float32),
          pltpu.VMEM((3, CB, h), jnp.float32),
          pltpu.VMEM((bpt, CB), jnp.int32),
          pltpu.VMEM((bpt, CB), jnp.int32),
          pltpu.VMEM((ept,), jnp.float32),
          pltpu.SemaphoreType.DMA,
          pltpu.SemaphoreType.DMA,
          pltpu.SemaphoreType.DMA,
          pltpu.SemaphoreType.DMA,
          pltpu.SemaphoreType.DMA,
          pltpu.SemaphoreType.DMA,
          pltpu.SemaphoreType.DMA,
      ],
  )
  def spmm(feat0_hbm, feat1_hbm, edge_hbm, w_hbm, out_hbm,
           acc, sfeat, rows, sidx, didx, wbuf, lsem, gs0, gs1, gs2,
           ss0, ss1, ss2):
    c = lax.axis_index("c")
    s = lax.axis_index("s")

    bb = s * bpt
    ld_s = pltpu.async_copy(edge_hbm.at[0, pl.ds(bb, bpt)], sidx, lsem)
    ld_d = pltpu.async_copy(edge_hbm.at[1, pl.ds(bb, bpt)], didx, lsem)
    ld_w = pltpu.async_copy(w_hbm.at[pl.ds(s * ept, ept)], wbuf, lsem)

    @pl.loop(0, CB)
    def _(r):
      for cb in range(h // 16):
        rows[0, r, pl.ds(cb * 16, 16)] = jnp.zeros((16,), jnp.float32)

    @pl.loop(0, rows_pt // CB)
    def _(j):
      rb = s * rows_pt + j * CB
      pltpu.sync_copy(rows.at[0], acc.at[pl.ds(rb, CB)])

      @pl.when(c == 0)
      def _():
        pltpu.sync_copy(feat0_hbm.at[pl.ds(rb, CB)],
                        sfeat.at[pl.ds(rb, CB)])

      @pl.when(c == 1)
      def _():
        pltpu.sync_copy(feat1_hbm.at[pl.ds(rb, CB)],
                        sfeat.at[pl.ds(rb, CB)])

    ld_s.wait()
    ld_d.wait()
    ld_w.wait()

    gsems = (gs0, gs1, gs2)
    ssems = (ss0, ss1, ss2)

    def start_gather(it, b):
      pltpu.async_copy(sfeat.at[sidx.at[it]], rows.at[b], gsems[b])

    def wait_gather(it, b):
      pltpu.make_async_copy(sfeat.at[sidx.at[it]], rows.at[b],
                            gsems[b]).wait()

    def scale(it, b):
      @pl.loop(0, CB // 16)
      def _(g):
        w16 = wbuf[pl.ds(it * CB + g * 16, 16)]
        ews = []
        for j in range(16):
          ews.append(lax.gather(
              w16, jnp.full((16, 1), j, jnp.int32),
              lax.GatherDimensionNumbers(
                  offset_dims=(), collapsed_slice_dims=(0,),
                  start_index_map=(0,)),
              slice_sizes=(1,),
              mode=lax.GatherScatterMode.PROMISE_IN_BOUNDS))
        for cb in range(h // 16):
          sl = pl.ds(cb * 16, 16)
          vals = [rows[b, g * 16 + j, sl] * ews[j] for j in range(16)]
          for j in range(16):
            rows[b, g * 16 + j, sl] = vals[j]

    def start_scatter(it, b):
      pltpu.async_copy(rows.at[b], acc.at[didx.at[it]], ssems[b], add=True)

    def drain_scatter(it, b):
      pltpu.make_async_copy(rows.at[b], acc.at[didx.at[it]],
                            ssems[b]).wait()

    plsc.subcore_barrier()
    start_gather(0, 0)
    start_gather(1, 1)

    n3 = bpt // 3

    @pl.loop(0, n3)
    def _(i3):
      for b in range(3):
        k = i3 * 3 + b
        wait_gather(k, b)
        scale(k, b)
        start_scatter(k, b)
        b2 = (b + 2) % 3
        if b == 0:
          @pl.when(i3 >= 1)
          def _():
            drain_scatter(k - 1, b2)
          start_gather(k + 2, b2)
        else:
          @pl.when(i3 < n3 - 1)
          def _():
            drain_scatter(k - 1, b2)
            start_gather(k + 2, b2)

    drain_scatter(bpt - 3, 0)
    drain_scatter(bpt - 2, 1)
    drain_scatter(bpt - 1, 2)
    plsc.subcore_barrier()

    @pl.loop(0, rows_pt // CB)
    def _(j):
      rb = s * rows_pt + j * CB
      if tanh_out:
        pltpu.sync_copy(acc.at[pl.ds(rb, CB)], rows.at[0])

        @pl.loop(0, CB // 8)
        def _(r8):
          for dr in range(8):
            r = r8 * 8 + dr
            for cb in range(h // 16):
              sl = pl.ds(cb * 16, 16)
              v = rows[0, r, sl]
              e2 = jnp.exp(v * 2.0)
              rows[0, r, sl] = 1.0 - 2.0 / (e2 + 1.0)

        pltpu.sync_copy(rows.at[0], out_hbm.at[c, pl.ds(rb, CB)])
      else:
        pltpu.sync_copy(acc.at[pl.ds(rb, CB)], out_hbm.at[c, pl.ds(rb, CB)])

  return spmm


def kernel(x, edge_index, edge_weight, W1, W2):
  n = x.shape[0]
  e = edge_index.shape[1]

  ept = -(-e // (NS * CB * 3)) * (CB * 3)
  e_pad = NS * ept
  n_pad = -(-n // (NS * CB)) * (NS * CB)

  eidx = edge_index.astype(jnp.int32)
  w = edge_weight.astype(jnp.float32)
  if e_pad != e:
    eidx = jnp.pad(eidx, ((0, 0), (0, e_pad - e)))
    w = jnp.concatenate([w, jnp.zeros((e_pad - e,), jnp.float32)])
  eidx = eidx.reshape(2, e_pad // CB, CB)

  if n_pad != n:
    x = jnp.concatenate(
        [x, jnp.zeros((n_pad - n, x.shape[1]), jnp.float32)])

  h1w = W1.shape[1]
  h2w = W2.shape[1]

  xw_l, xw_r = _tc_matmul_split(x, W1)
  p = _make_spmm_col(n_pad, h1w // 2, ept, False)(
      xw_l, xw_r, eidx, w)
  y2_l, y2_r = _tc_relu_matmul_split(p, W2)
  q = _make_spmm_col(n_pad, h2w // 2, ept, True)(
      y2_l, y2_r, eidx, w)
  return jnp.concatenate([q[0], q[1]], axis=1)[:n]


# speedup vs baseline: 1.1449x; 1.0535x over previous
import functools

import jax
import jax.numpy as jnp
from jax import lax
from jax.experimental import pallas as pl
from jax.experimental.pallas import tpu as pltpu
from jax.experimental.pallas import tpu_sc as plsc

NC = 2
NS = 16
CB = 128


def _tc_matmul_split(x, w):
  m = x.shape[0]
  h = w.shape[1]
  hh = h // 2

  def body(x_ref, w_ref, o1_ref, o2_ref):
    xw = jnp.dot(x_ref[...], w_ref[...], preferred_element_type=jnp.float32)
    o1_ref[...] = xw[:, :hh]
    o2_ref[...] = xw[:, hh:]

  return pl.pallas_call(
      body,
      out_shape=[jax.ShapeDtypeStruct((m, hh), jnp.float32),
                 jax.ShapeDtypeStruct((m, hh), jnp.float32)],
  )(x, w)


def _tc_relu_matmul_split(p, w):
  _, n_pad, k = p.shape
  h = w.shape[1]
  hh = h // 2

  def body(p_ref, w_ref, o1_ref, o2_ref):
    h1l = jnp.maximum(p_ref[0], 0.0)
    h1r = jnp.maximum(p_ref[1], 0.0)
    y = (jnp.dot(h1l, w_ref[:k], preferred_element_type=jnp.float32)
         + jnp.dot(h1r, w_ref[k:], preferred_element_type=jnp.float32))
    o1_ref[...] = y[:, :hh]
    o2_ref[...] = y[:, hh:]

  return pl.pallas_call(
      body,
      out_shape=[jax.ShapeDtypeStruct((n_pad, hh), jnp.float32),
                 jax.ShapeDtypeStruct((n_pad, hh), jnp.float32)],
  )(p, w)


@functools.lru_cache(maxsize=None)
def _make_spmm_col(n_pad, h, ept, tanh_out):
  rows_pt = n_pad // NS
  bpt = ept // CB
  mesh = plsc.VectorSubcoreMesh(core_axis_name="c", subcore_axis_name="s")

  @functools.partial(
      pl.kernel,
      out_type=(jax.ShapeDtypeStruct((n_pad, NC * h), jnp.float32)
                if tanh_out else
                jax.ShapeDtypeStruct((NC, n_pad, h), jnp.float32)),
      mesh=mesh,
      compiler_params=pltpu.CompilerParams(use_tc_tiling_on_sc=False),
      scratch_types=[
          pltpu.VMEM_SHARED((n_pad, h), jnp.float32),
          pltpu.VMEM_SHARED((n_pad, h), jnp.float32),
          pltpu.VMEM((3, CB, h), jnp.float32),
          pltpu.VMEM((bpt, CB), jnp.int32),
          pltpu.VMEM((bpt, CB), jnp.int32),
          pltpu.VMEM((ept,), jnp.float32),
          pltpu.SemaphoreType.DMA,
          pltpu.SemaphoreType.DMA,
          pltpu.SemaphoreType.DMA,
          pltpu.SemaphoreType.DMA,
          pltpu.SemaphoreType.DMA,
          pltpu.SemaphoreType.DMA,
          pltpu.SemaphoreType.DMA,
      ],
  )
  def spmm(feat0_hbm, feat1_hbm, edge_hbm, w_hbm, out_hbm,
           acc, sfeat, rows, sidx, didx, wbuf, lsem, gs0, gs1, gs2,
           ss0, ss1, ss2):
    c = lax.axis_index("c")
    s = lax.axis_index("s")

    bb = s * bpt
    ld_s = pltpu.async_copy(edge_hbm.at[0, pl.ds(bb, bpt)], sidx, lsem)
    ld_d = pltpu.async_copy(edge_hbm.at[1, pl.ds(bb, bpt)], didx, lsem)
    ld_w = pltpu.async_copy(w_hbm.at[pl.ds(s * ept, ept)], wbuf, lsem)

    @pl.loop(0, CB)
    def _(r):
      for cb in range(h // 16):
        rows[0, r, pl.ds(cb * 16, 16)] = jnp.zeros((16,), jnp.float32)

    @pl.loop(0, rows_pt // CB)
    def _(j):
      rb = s * rows_pt + j * CB
      pltpu.sync_copy(rows.at[0], acc.at[pl.ds(rb, CB)])

      @pl.when(c == 0)
      def _():
        pltpu.sync_copy(feat0_hbm.at[pl.ds(rb, CB)],
                        sfeat.at[pl.ds(rb, CB)])

      @pl.when(c == 1)
      def _():
        pltpu.sync_copy(feat1_hbm.at[pl.ds(rb, CB)],
                        sfeat.at[pl.ds(rb, CB)])

    ld_s.wait()
    ld_d.wait()
    ld_w.wait()

    gsems = (gs0, gs1, gs2)
    ssems = (ss0, ss1, ss2)

    def start_gather(it, b):
      pltpu.async_copy(sfeat.at[sidx.at[it]], rows.at[b], gsems[b])

    def wait_gather(it, b):
      pltpu.make_async_copy(sfeat.at[sidx.at[it]], rows.at[b],
                            gsems[b]).wait()

    def scale(it, b):
      @pl.loop(0, CB // 16)
      def _(g):
        w16 = wbuf[pl.ds(it * CB + g * 16, 16)]
        ews = []
        for j in range(16):
          ews.append(lax.gather(
              w16, jnp.full((16, 1), j, jnp.int32),
              lax.GatherDimensionNumbers(
                  offset_dims=(), collapsed_slice_dims=(0,),
                  start_index_map=(0,)),
              slice_sizes=(1,),
              mode=lax.GatherScatterMode.PROMISE_IN_BOUNDS))
        for cb in range(h // 16):
          sl = pl.ds(cb * 16, 16)
          vals = [rows[b, g * 16 + j, sl] * ews[j] for j in range(16)]
          for j in range(16):
            rows[b, g * 16 + j, sl] = vals[j]

    def start_scatter(it, b):
      pltpu.async_copy(rows.at[b], acc.at[didx.at[it]], ssems[b], add=True)

    def drain_scatter(it, b):
      pltpu.make_async_copy(rows.at[b], acc.at[didx.at[it]],
                            ssems[b]).wait()

    plsc.subcore_barrier()
    start_gather(0, 0)
    start_gather(1, 1)

    n3 = bpt // 3

    @pl.loop(0, n3)
    def _(i3):
      for b in range(3):
        k = i3 * 3 + b
        wait_gather(k, b)
        scale(k, b)
        start_scatter(k, b)
        b2 = (b + 2) % 3
        if b == 0:
          @pl.when(i3 >= 1)
          def _():
            drain_scatter(k - 1, b2)
          start_gather(k + 2, b2)
        else:
          @pl.when(i3 < n3 - 1)
          def _():
            drain_scatter(k - 1, b2)
            start_gather(k + 2, b2)

    drain_scatter(bpt - 3, 0)
    drain_scatter(bpt - 2, 1)
    drain_scatter(bpt - 1, 2)
    plsc.subcore_barrier()

    @pl.loop(0, rows_pt // CB)
    def _(j):
      rb = s * rows_pt + j * CB
      if tanh_out:
        pltpu.sync_copy(acc.at[pl.ds(rb, CB)], rows.at[0])

        @pl.loop(0, CB // 8)
        def _(r8):
          for dr in range(8):
            r = r8 * 8 + dr
            for cb in range(h // 16):
              sl = pl.ds(cb * 16, 16)
              v = rows[0, r, sl]
              e2 = jnp.exp(v * 2.0)
              rows[0, r, sl] = 1.0 - 2.0 / (e2 + 1.0)

        pltpu.sync_copy(rows.at[0],
                        out_hbm.at[pl.ds(rb, CB), pl.ds(c * h, h)])
      else:
        pltpu.sync_copy(acc.at[pl.ds(rb, CB)], out_hbm.at[c, pl.ds(rb, CB)])

  return spmm


def kernel(x, edge_index, edge_weight, W1, W2):
  n = x.shape[0]
  e = edge_index.shape[1]

  ept = -(-e // (NS * CB * 3)) * (CB * 3)
  e_pad = NS * ept
  n_pad = -(-n // (NS * CB)) * (NS * CB)

  eidx = edge_index.astype(jnp.int32)
  w = edge_weight.astype(jnp.float32)
  if e_pad != e:
    eidx = jnp.pad(eidx, ((0, 0), (0, e_pad - e)))
    w = jnp.concatenate([w, jnp.zeros((e_pad - e,), jnp.float32)])
  eidx = eidx.reshape(2, e_pad // CB, CB)

  if n_pad != n:
    x = jnp.concatenate(
        [x, jnp.zeros((n_pad - n, x.shape[1]), jnp.float32)])

  h1w = W1.shape[1]
  h2w = W2.shape[1]

  xw_l, xw_r = _tc_matmul_split(x, W1)
  p = _make_spmm_col(n_pad, h1w // 2, ept, False)(
      xw_l, xw_r, eidx, w)
  y2_l, y2_r = _tc_relu_matmul_split(p, W2)
  q = _make_spmm_col(n_pad, h2w // 2, ept, True)(
      y2_l, y2_r, eidx, w)
  return q[:n]
